# Initial kernel scaffold; baseline (speedup 1.0000x reference)
import jax
import jax.numpy as jnp
from jax import lax
from jax.experimental import pallas as pl
from jax.experimental.pallas import tpu as pltpu
from jax.experimental.pallas import tpu_sc as plsc

N = 10000
E = 320000
D = 128
B = 128
K = 16
L = 3

NSUB = 16
CHUNK = 128
NCHUNK = 160
EPT = CHUNK * NCHUNK
NPAD = 10016
DUMP = N
STRIPE = NPAD // NSUB
NBUF = 4
LA = 2

BLK = 2000
GRID = N // BLK

_mesh = plsc.VectorSubcoreMesh(core_axis_name="c", subcore_axis_name="s")


def _deg_body(dst1, dst2, oini, out1, out2, idx_d, ones_b, acc,
              s0, s1, s2, s3):
    c = lax.axis_index("c")
    s = lax.axis_index("s")
    sems = [s0, s1, s2, s3]

    def run(dst_r, out_r):
        pltpu.sync_copy(dst_r.at[s], idx_d)
        pltpu.sync_copy(oini.at[pl.ds(0, CHUNK)], ones_b)
        pltpu.sync_copy(oini.at[pl.ds(s * STRIPE, STRIPE)],
                        acc.at[pl.ds(s * STRIPE, STRIPE)])
        plsc.subcore_barrier()

        def outer(o, carry):
            for b in range(NBUF):
                t = o * NBUF + b

                @pl.when(t >= NBUF)
                def _():
                    pltpu.make_async_copy(
                        ones_b, acc.at[idx_d.at[0]], sems[b]).wait()

                pltpu.async_copy(ones_b, acc.at[idx_d.at[t]], sems[b],
                                 add=True)
            return carry

        lax.fori_loop(0, NCHUNK // NBUF, outer, 0)
        for b in range(NBUF):
            pltpu.make_async_copy(ones_b, acc.at[idx_d.at[0]], sems[b]).wait()
        plsc.subcore_barrier()
        pltpu.sync_copy(acc.at[pl.ds(s * STRIPE, STRIPE)],
                        out_r.at[pl.ds(s * STRIPE, STRIPE)])

    @pl.when(c == 0)
    def _():
        run(dst1, out1)

    @pl.when(c == 1)
    def _():
        run(dst2, out2)


def _sc_deg(dst1p, dst2p, oini):
    fn = pl.kernel(
        _deg_body,
        out_type=(jax.ShapeDtypeStruct((NPAD, 16), jnp.float32),
                  jax.ShapeDtypeStruct((NPAD, 16), jnp.float32)),
        mesh=_mesh,
        scratch_types=[
            pltpu.VMEM((NCHUNK, CHUNK), jnp.int32),
            pltpu.VMEM((CHUNK, 16), jnp.float32),
            pltpu.VMEM_SHARED((NPAD, 16), jnp.float32),
            pltpu.SemaphoreType.DMA,
            pltpu.SemaphoreType.DMA,
            pltpu.SemaphoreType.DMA,
            pltpu.SemaphoreType.DMA,
        ],
    )
    return fn(dst1p, dst2p, oini)


def _msg_body(src1, dst1, src2, dst2, g1, g2, zini, out1, out2,
              idx_s, idx_d, rows, acc,
              sg0, sg1, sg2, sg3, ss0, ss1, ss2, ss3):
    c = lax.axis_index("c")
    s = lax.axis_index("s")
    sem_g = [sg0, sg1, sg2, sg3]
    sem_s = [ss0, ss1, ss2, ss3]

    def run(src_r, dst_r, g_r, out_r):
        pltpu.sync_copy(src_r.at[s], idx_s)
        pltpu.sync_copy(dst_r.at[s], idx_d)
        pltpu.sync_copy(zini.at[pl.ds(s * STRIPE, STRIPE)],
                        acc.at[pl.ds(s * STRIPE, STRIPE)])
        plsc.subcore_barrier()

        def wait_gather(b):
            pltpu.make_async_copy(g_r.at[idx_s.at[0]], rows.at[b],
                                  sem_g[b]).wait()

        def wait_scatter(b):
            pltpu.make_async_copy(rows.at[b], acc.at[idx_d.at[0]],
                                  sem_s[b]).wait()

        def outer(o, carry):
            for b in range(NBUF):
                t = o * NBUF + b

                @pl.when(t < NCHUNK)
                def _():
                    @pl.when(t >= NBUF)
                    def _():
                        wait_scatter(b)

                    pltpu.async_copy(g_r.at[idx_s.at[t]], rows.at[b],
                                     sem_g[b])

                u = t - LA
                bu = (b - LA) % NBUF

                @pl.when(jnp.logical_and(u >= 0, u < NCHUNK))
                def _():
                    wait_gather(bu)
                    pltpu.async_copy(rows.at[bu], acc.at[idx_d.at[u]],
                                     sem_s[bu], add=True)
            return carry

        nslots = NCHUNK + LA
        lax.fori_loop(0, (nslots + NBUF - 1) // NBUF, outer, 0)
        for b in range(NBUF):
            wait_scatter(b)
        plsc.subcore_barrier()
        pltpu.sync_copy(acc.at[pl.ds(s * STRIPE, STRIPE)],
                        out_r.at[pl.ds(s * STRIPE, STRIPE)])

    @pl.when(c == 0)
    def _():
        run(src1, dst1, g1, out1)

    @pl.when(c == 1)
    def _():
        run(src2, dst2, g2, out2)


def _sc_msg(src1p, dst1p, src2p, dst2p, g1, g2, zini):
    fn = pl.kernel(
        _msg_body,
        out_type=(jax.ShapeDtypeStruct((NPAD, D), jnp.float32),
                  jax.ShapeDtypeStruct((NPAD, D), jnp.float32)),
        mesh=_mesh,
        scratch_types=[
            pltpu.VMEM((NCHUNK, CHUNK), jnp.int32),
            pltpu.VMEM((NCHUNK, CHUNK), jnp.int32),
            pltpu.VMEM((NBUF, CHUNK, D), jnp.float32),
            pltpu.VMEM_SHARED((NPAD, D), jnp.float32),
            pltpu.SemaphoreType.DMA, pltpu.SemaphoreType.DMA,
            pltpu.SemaphoreType.DMA, pltpu.SemaphoreType.DMA,
            pltpu.SemaphoreType.DMA, pltpu.SemaphoreType.DMA,
            pltpu.SemaphoreType.DMA, pltpu.SemaphoreType.DMA,
        ],
    )
    return fn(src1p, dst1p, src2p, dst2p, g1, g2, zini)


_tc_params = pltpu.CompilerParams(dimension_semantics=("arbitrary",))


def _pre_body(x_ref, deg_ref, w_ref, g_ref):
    dinv = lax.rsqrt(jnp.maximum(deg_ref[...], 1.0))
    g_ref[...] = dinv * jnp.dot(x_ref[...], w_ref[...],
                                preferred_element_type=jnp.float32)


def _tc_pre(x, deg, W):
    return pl.pallas_call(
        _pre_body,
        grid=(GRID,),
        in_specs=[
            pl.BlockSpec((BLK, D), lambda i: (i, 0)),
            pl.BlockSpec((BLK, 1), lambda i: (i, 0)),
            pl.BlockSpec((D, D), lambda i: (0, 0)),
        ],
        out_specs=pl.BlockSpec((BLK, D), lambda i: (i, 0)),
        out_shape=jax.ShapeDtypeStruct((N, D), jnp.float32),
        compiler_params=_tc_params,
    )(x, deg, W)


def _c1_body_factory(has_next, first):
    def body(*refs):
        if has_next:
            (s_ref, g_ref, deg_ref, batr_ref, b_ref, wc_ref, wp_ref,
             wn_ref) = refs[:8]
            outs = refs[8:]
        else:
            s_ref, g_ref, deg_ref, batr_ref, b_ref, wc_ref, wp_ref = refs[:7]
            outs = refs[7:]
        xn_ref = outs[0]
        idx = 1
        if has_next:
            gn_ref = outs[idx]
            idx += 1
        segc_ref = outs[idx]
        segp_ref = outs[idx + 1]
        idx += 2
        if first:
            cnt_ref = outs[idx]

        i = pl.program_id(0)
        dinv = lax.rsqrt(jnp.maximum(deg_ref[...], 1.0))
        xn = jnp.maximum((s_ref[...] + g_ref[...]) * dinv + b_ref[...], 0.0)
        xn_ref[...] = xn
        if has_next:
            gn_ref[...] = dinv * jnp.dot(xn, wn_ref[...],
                                         preferred_element_type=jnp.float32)
        batr = batr_ref[0:1, :]
        iota_b = lax.broadcasted_iota(jnp.int32, (B, 1), 0)
        mbt = (batr == iota_b).astype(jnp.float32)
        hc = jnp.dot(xn, wc_ref[...], preferred_element_type=jnp.float32)
        hp = jnp.dot(xn, wp_ref[...], preferred_element_type=jnp.float32)
        cc = jnp.dot(mbt, hc, preferred_element_type=jnp.float32)
        cp = jnp.dot(mbt, hp, preferred_element_type=jnp.float32)

        @pl.when(i == 0)
        def _():
            segc_ref[...] = cc
            segp_ref[...] = cp
            if first:
                cnt_ref[...] = jnp.sum(mbt, axis=1, keepdims=True)

        @pl.when(i > 0)
        def _():
            segc_ref[...] += cc
            segp_ref[...] += cp
            if first:
                cnt_ref[...] += jnp.sum(mbt, axis=1, keepdims=True)

    return body


def _tc_c1(S, g, deg, batr8, b_gcn, Wc, Wp, Wn, first):
    has_next = Wn is not None
    in_specs = [
        pl.BlockSpec((BLK, D), lambda i: (i, 0)),
        pl.BlockSpec((BLK, D), lambda i: (i, 0)),
        pl.BlockSpec((BLK, 1), lambda i: (i, 0)),
        pl.BlockSpec((8, BLK), lambda i: (0, i)),
        pl.BlockSpec((1, D), lambda i: (0, 0)),
        pl.BlockSpec((D, D), lambda i: (0, 0)),
        pl.BlockSpec((D, D), lambda i: (0, 0)),
    ]
    args = [S, g, deg, batr8, b_gcn, Wc, Wp]
    if has_next:
        in_specs.append(pl.BlockSpec((D, D), lambda i: (0, 0)))
        args.append(Wn)
    out_specs = [pl.BlockSpec((BLK, D), lambda i: (i, 0))]
    out_shape = [jax.ShapeDtypeStruct((N, D), jnp.float32)]
    if has_next:
        out_specs.append(pl.BlockSpec((BLK, D), lambda i: (i, 0)))
        out_shape.append(jax.ShapeDtypeStruct((N, D), jnp.float32))
    out_specs += [pl.BlockSpec((B, D), lambda i: (0, 0)),
                  pl.BlockSpec((B, D), lambda i: (0, 0))]
    out_shape += [jax.ShapeDtypeStruct((B, D), jnp.float32),
                  jax.ShapeDtypeStruct((B, D), jnp.float32)]
    if first:
        out_specs.append(pl.BlockSpec((B, 1), lambda i: (0, 0)))
        out_shape.append(jax.ShapeDtypeStruct((B, 1), jnp.float32))
    return pl.pallas_call(
        _c1_body_factory(has_next, first),
        grid=(GRID,),
        in_specs=in_specs,
        out_specs=out_specs,
        out_shape=out_shape,
        compiler_params=_tc_params,
    )(*args)


def _c2_body(xn_ref, batc_ref, batr_ref, segc_ref, segp_ref, cnt_ref,
             pc_ref, pp_ref):
    i = pl.program_id(0)
    cntm = jnp.maximum(cnt_ref[...], 1.0)
    ctxc = jnp.tanh(segc_ref[...] / cntm)
    ctxp = jnp.tanh(segp_ref[...] / cntm)
    xn = xn_ref[...]
    batc = batc_ref[...]
    iota_r = lax.broadcasted_iota(jnp.int32, (1, B), 1)
    mb = (batc == iota_r).astype(jnp.float32)
    batr = batr_ref[0:1, :]
    iota_b = lax.broadcasted_iota(jnp.int32, (B, 1), 0)
    mbt = (batr == iota_b).astype(jnp.float32)
    ec = jnp.dot(mb, ctxc, preferred_element_type=jnp.float32)
    ep = jnp.dot(mb, ctxp, preferred_element_type=jnp.float32)
    attc = jax.nn.sigmoid(jnp.sum(xn * ec, axis=1, keepdims=True))
    attp = jax.nn.sigmoid(jnp.sum(xn * ep, axis=1, keepdims=True))
    vc = jnp.dot(mbt, xn * attc, preferred_element_type=jnp.float32)
    vp = jnp.dot(mbt, xn * attp, preferred_element_type=jnp.float32)

    @pl.when(i == 0)
    def _():
        pc_ref[...] = vc
        pp_ref[...] = vp

    @pl.when(i > 0)
    def _():
        pc_ref[...] += vc
        pp_ref[...] += vp


def _tc_c2(xn, batc, batr8, segc, segp, cnt):
    return pl.pallas_call(
        _c2_body,
        grid=(GRID,),
        in_specs=[
            pl.BlockSpec((BLK, D), lambda i: (i, 0)),
            pl.BlockSpec((BLK, 1), lambda i: (i, 0)),
            pl.BlockSpec((8, BLK), lambda i: (0, i)),
            pl.BlockSpec((B, D), lambda i: (0, 0)),
            pl.BlockSpec((B, D), lambda i: (0, 0)),
            pl.BlockSpec((B, 1), lambda i: (0, 0)),
        ],
        out_specs=[pl.BlockSpec((B, D), lambda i: (0, 0)),
                   pl.BlockSpec((B, D), lambda i: (0, 0))],
        out_shape=[jax.ShapeDtypeStruct((B, D), jnp.float32),
                   jax.ShapeDtypeStruct((B, D), jnp.float32)],
        compiler_params=_tc_params,
    )(xn, batc, batr8, segc, segp, cnt)


def _cos_rows(a, b):
    an = a / jnp.maximum(jnp.sqrt(jnp.sum(a * a, axis=1, keepdims=True)),
                         1e-8)
    bn = b / jnp.maximum(jnp.sqrt(jnp.sum(b * b, axis=1, keepdims=True)),
                         1e-8)
    return jnp.sum(an * bn, axis=1, keepdims=True)


def _corr_rows(a, b):
    ac = a - jnp.mean(a, axis=0, keepdims=True)
    bc = b - jnp.mean(b, axis=0, keepdims=True)
    c = _cos_rows(ac, bc)
    return c * c


def _final_body(pc10, pp10, pc11, pp11, pc12, pp12,
                pc20, pp20, pc21, pp21, pc22, pp22,
                wt_ref, vt_ref, bntn_ref, ws1_ref, bs1_ref, ws2_ref, bs2_ref,
                score_ref, loss_ref):
    com1 = [pc10[...], pc11[...], pc12[...]]
    pri1 = [pp10[...], pp11[...], pp12[...]]
    com2 = [pc20[...], pc21[...], pc22[...]]
    pri2 = [pp20[...], pp21[...], pp22[...]]
    f1 = jnp.concatenate([com1[2], pri1[2]], axis=1)
    f2 = jnp.concatenate([com2[2], pri2[2]], axis=1)
    cols = []
    for k in range(K):
        t = jnp.dot(f1, wt_ref[k], preferred_element_type=jnp.float32)
        cols.append(jnp.sum(t * f2, axis=1, keepdims=True))
    scoring = jnp.concatenate(cols, axis=1)
    blk = jnp.dot(jnp.concatenate([f1, f2], axis=1), vt_ref[...],
                  preferred_element_type=jnp.float32)
    sact = jnp.maximum(scoring + blk + bntn_ref[...], 0.0)
    h1 = jnp.maximum(jnp.dot(sact, ws1_ref[...],
                             preferred_element_type=jnp.float32)
                     + bs1_ref[...], 0.0)
    pre = jnp.sum(h1 * ws2_ref[...], axis=1, keepdims=True) + bs2_ref[...]
    score_ref[...] = jax.nn.sigmoid(pre)

    cor_sum = jnp.zeros((1, 1), jnp.float32)
    lratio_sum = jnp.zeros((1, 1), jnp.float32)
    for i in range(L):
        cor1 = _corr_rows(com1[i], pri1[i])
        cor2 = _corr_rows(com2[i], pri2[i])
        cor_sum = cor_sum + jnp.sum(cor1 + cor2, axis=0, keepdims=True)
        sim_com = jnp.exp(_cos_rows(com1[i], com2[i]))
        sim_pri = jnp.exp(_cos_rows(com2[i], pri2[i]))
        lr = jnp.log(sim_com / (sim_com + sim_pri))
        lratio_sum = lratio_sum + jnp.sum(lr, axis=0, keepdims=True)
    loss_ref[...] = (-lratio_sum / (L * B)) + 0.5 * (cor_sum / (L * B))


def _tc_final(pools, Wt, Vt, b_ntn, W_s1, b_s1, W_s2r, b_s2):
    def full(shp):
        return pl.BlockSpec(shp, lambda: tuple(0 for _ in shp))

    in_specs = [full((B, D)) for _ in range(12)] + [
        full((K, 2 * D, 2 * D)), full((4 * D, K)), full((1, K)),
        full((K, K)), full((1, K)), full((1, K)), full((1, 1)),
    ]
    return pl.pallas_call(
        _final_body,
        grid=(),
        in_specs=in_specs,
        out_specs=[full((B, 1)), full((1, 1))],
        out_shape=[jax.ShapeDtypeStruct((B, 1), jnp.float32),
                   jax.ShapeDtypeStruct((1, 1), jnp.float32)],
    )(*pools, Wt, Vt, b_ntn, W_s1, b_s1, W_s2r, b_s2)


def _pad_edges(ei):
    pads = EPT * NSUB - E
    src = jnp.concatenate([ei[0], jnp.zeros((pads,), jnp.int32)])
    dst = jnp.concatenate([ei[1], jnp.full((pads,), DUMP, jnp.int32)])
    return (src.reshape(NSUB, NCHUNK, CHUNK),
            dst.reshape(NSUB, NCHUNK, CHUNK))


def kernel(x1, x2, edge_index1, edge_index2, batch1, batch2,
           W_gcn0, b_gcn0, W_com0, W_pri0,
           W_gcn1, b_gcn1, W_com1, W_pri1,
           W_gcn2, b_gcn2, W_com2, W_pri2,
           W_ntn, V_ntn, b_ntn, W_s1, b_s1, W_s2, b_s2):
    src1p, dst1p = _pad_edges(edge_index1)
    src2p, dst2p = _pad_edges(edge_index2)
    oini = jnp.ones((NPAD, 16), jnp.float32)
    zini = jnp.zeros((NPAD, D), jnp.float32)

    degw1, degw2 = _sc_deg(dst1p, dst2p, oini)
    deg1 = degw1[:N, 0:1]
    deg2 = degw2[:N, 0:1]

    batc1 = batch1.reshape(N, 1)
    batc2 = batch2.reshape(N, 1)
    batr1 = jnp.broadcast_to(batch1.reshape(1, N), (8, N))
    batr2 = jnp.broadcast_to(batch2.reshape(1, N), (8, N))

    Wg = [W_gcn0, W_gcn1, W_gcn2]
    bg = [b_gcn0.reshape(1, D), b_gcn1.reshape(1, D), b_gcn2.reshape(1, D)]
    Wc = [W_com0, W_com1, W_com2]
    Wp = [W_pri0, W_pri1, W_pri2]

    g1 = _tc_pre(x1, deg1, Wg[0])
    g2 = _tc_pre(x2, deg2, Wg[0])

    pools1 = []
    pools2 = []
    cnt1 = cnt2 = None
    for i in range(L):
        S1p, S2p = _sc_msg(src1p, dst1p, src2p, dst2p, g1, g2, zini)
        S1 = S1p[:N]
        S2 = S2p[:N]
        Wn = Wg[i + 1] if i < L - 1 else None
        r1 = _tc_c1(S1, g1, deg1, batr1, bg[i], Wc[i], Wp[i], Wn,
                    first=(i == 0))
        r2 = _tc_c1(S2, g2, deg2, batr2, bg[i], Wc[i], Wp[i], Wn,
                    first=(i == 0))
        if i == 0:
            x1n, g1, segc1, segp1, cnt1 = r1
            x2n, g2, segc2, segp2, cnt2 = r2
        elif i < L - 1:
            x1n, g1, segc1, segp1 = r1
            x2n, g2, segc2, segp2 = r2
        else:
            x1n, segc1, segp1 = r1
            x2n, segc2, segp2 = r2
        pc1, pp1 = _tc_c2(x1n, batc1, batr1, segc1, segp1, cnt1)
        pc2, pp2 = _tc_c2(x2n, batc2, batr2, segc2, segp2, cnt2)
        pools1.append((pc1, pp1))
        pools2.append((pc2, pp2))

    pools = [pools1[0][0], pools1[0][1], pools1[1][0], pools1[1][1],
             pools1[2][0], pools1[2][1],
             pools2[0][0], pools2[0][1], pools2[1][0], pools2[1][1],
             pools2[2][0], pools2[2][1]]
    Wt = jnp.transpose(W_ntn, (2, 0, 1))
    Vt = V_ntn.T
    score, loss = _tc_final(pools, Wt, Vt, b_ntn.reshape(1, K),
                            W_s1, b_s1.reshape(1, K),
                            W_s2.reshape(1, K), b_s2.reshape(1, 1))
    return score, loss[0, 0]


# speedup vs baseline: 9.6939x; 9.6939x over previous
import jax
import jax.numpy as jnp
from jax import lax
from jax.experimental import pallas as pl
from jax.experimental.pallas import tpu as pltpu
from jax.experimental.pallas import tpu_sc as plsc

N = 10000
E = 320000
D = 128
B = 128
K = 16
L = 3

NSUB = 16
CHUNK = 128
NCHUNK = 160
EPT = CHUNK * NCHUNK
NPAD = 10112
DUMP = N
STRIPE = NPAD // NSUB
NBUF = 4
LA = 2

BLK = 2000
GRID = N // BLK

def _mesh():
    return plsc.VectorSubcoreMesh(core_axis_name="c", subcore_axis_name="s")


def _deg_body(dst1, dst2, oini, out1, out2, idx_d, ones_b, acc,
              s0, s1, s2, s3):
    c = lax.axis_index("c")
    s = lax.axis_index("s")
    sems = [s0, s1, s2, s3]

    def run(dst_r, out_r):
        pltpu.sync_copy(dst_r.at[s], idx_d)
        pltpu.sync_copy(oini.at[pl.ds(0, CHUNK)], ones_b)
        pltpu.sync_copy(oini.at[pl.ds(s * STRIPE, STRIPE)],
                        acc.at[pl.ds(s * STRIPE, STRIPE)])
        plsc.subcore_barrier()

        def outer(o, carry):
            for b in range(NBUF):
                t = o * NBUF + b

                @pl.when(t >= NBUF)
                def _():
                    pltpu.make_async_copy(
                        ones_b, acc.at[idx_d.at[0]], sems[b]).wait()

                pltpu.async_copy(ones_b, acc.at[idx_d.at[t]], sems[b],
                                 add=True)
            return carry

        lax.fori_loop(0, NCHUNK // NBUF, outer, 0)
        for b in range(NBUF):
            pltpu.make_async_copy(ones_b, acc.at[idx_d.at[0]], sems[b]).wait()
        plsc.subcore_barrier()
        pltpu.sync_copy(acc.at[pl.ds(s * STRIPE, STRIPE)],
                        out_r.at[pl.ds(s * STRIPE, STRIPE)])

    @pl.when(c == 0)
    def _():
        run(dst1, out1)

    @pl.when(c == 1)
    def _():
        run(dst2, out2)


def _sc_deg(dst1p, dst2p, oini):
    fn = pl.kernel(
        _deg_body,
        out_type=(jax.ShapeDtypeStruct((NPAD, 16), jnp.float32),
                  jax.ShapeDtypeStruct((NPAD, 16), jnp.float32)),
        mesh=_mesh(),
        scratch_types=[
            pltpu.VMEM((NCHUNK, CHUNK), jnp.int32),
            pltpu.VMEM((CHUNK, 16), jnp.float32),
            pltpu.VMEM_SHARED((NPAD, 16), jnp.float32),
            pltpu.SemaphoreType.DMA,
            pltpu.SemaphoreType.DMA,
            pltpu.SemaphoreType.DMA,
            pltpu.SemaphoreType.DMA,
        ],
    )
    return fn(dst1p, dst2p, oini)


IB = 16
NGRP = NCHUNK // IB
NIB = 3


def _msg_body(src1, dst1, src2, dst2, g1, g2, zini, out1, out2,
              idx_s, idx_d, rows, acc,
              sg0, sg1, ss0, ss1, si):
    c = lax.axis_index("c")
    s = lax.axis_index("s")
    sem_g = [sg0, sg1]
    sem_s = [ss0, ss1]

    def run(src_r, dst_r, g_r, out_r):
        pltpu.sync_copy(zini.at[pl.ds(s * STRIPE, STRIPE)],
                        acc.at[pl.ds(s * STRIPE, STRIPE)])
        pltpu.async_copy(src_r.at[s].at[pl.ds(0, IB)], idx_s.at[0], si)
        pltpu.async_copy(dst_r.at[s].at[pl.ds(0, IB)], idx_d.at[0], si)
        plsc.subcore_barrier()

        def wait_gather(b):
            pltpu.make_async_copy(g_r.at[idx_s.at[0].at[0]], rows.at[b],
                                  sem_g[b]).wait()

        def wait_scatter(b):
            pltpu.make_async_copy(rows.at[b], acc.at[idx_d.at[0].at[0]],
                                  sem_s[b]).wait()

        def wait_idx():
            pltpu.make_async_copy(src_r.at[s].at[pl.ds(0, IB)],
                                  idx_s.at[0], si).wait()

        def outer(g, carry):
            gb = g % NIB
            gbn = (g + 1) % NIB
            gbp = (g - 1) % NIB
            wait_idx()
            wait_idx()

            @pl.when(g + 1 < NGRP)
            def _():
                pltpu.async_copy(src_r.at[s].at[pl.ds((g + 1) * IB, IB)],
                                 idx_s.at[gbn], si)
                pltpu.async_copy(dst_r.at[s].at[pl.ds((g + 1) * IB, IB)],
                                 idx_d.at[gbn], si)

            for t in range(IB):
                u = g * IB + t
                b = t % 2

                @pl.when(u >= 2)
                def _():
                    wait_scatter(b)

                pltpu.async_copy(g_r.at[idx_s.at[gb].at[t]], rows.at[b],
                                 sem_g[b])

                @pl.when(u >= 1)
                def _():
                    wait_gather(1 - b)
                    if t > 0:
                        pidx = idx_d.at[gb].at[t - 1]
                    else:
                        pidx = idx_d.at[gbp].at[IB - 1]
                    pltpu.async_copy(rows.at[1 - b], acc.at[pidx],
                                     sem_s[1 - b], add=True)
            return carry

        lax.fori_loop(0, NGRP, outer, 0)
        bl = (IB - 1) % 2
        wait_gather(bl)
        pltpu.async_copy(rows.at[bl],
                         acc.at[idx_d.at[(NGRP - 1) % NIB].at[IB - 1]],
                         sem_s[bl], add=True)
        wait_scatter(0)
        wait_scatter(1)
        plsc.subcore_barrier()
        pltpu.sync_copy(acc.at[pl.ds(s * STRIPE, STRIPE)],
                        out_r.at[pl.ds(s * STRIPE, STRIPE)])

    @pl.when(c == 0)
    def _():
        run(src1, dst1, g1, out1)

    @pl.when(c == 1)
    def _():
        run(src2, dst2, g2, out2)


def _sc_msg(src1p, dst1p, src2p, dst2p, g1, g2, zini):
    fn = pl.kernel(
        _msg_body,
        out_type=(jax.ShapeDtypeStruct((NPAD, D), jnp.float32),
                  jax.ShapeDtypeStruct((NPAD, D), jnp.float32)),
        mesh=_mesh(),
        scratch_types=[
            pltpu.VMEM((NIB, IB, CHUNK), jnp.int32),
            pltpu.VMEM((NIB, IB, CHUNK), jnp.int32),
            pltpu.VMEM((2, CHUNK, D), jnp.float32),
            pltpu.VMEM_SHARED((NPAD, D), jnp.float32),
            pltpu.SemaphoreType.DMA, pltpu.SemaphoreType.DMA,
            pltpu.SemaphoreType.DMA, pltpu.SemaphoreType.DMA,
            pltpu.SemaphoreType.DMA,
        ],
    )
    return fn(src1p, dst1p, src2p, dst2p, g1, g2, zini)


_tc_params = pltpu.CompilerParams(dimension_semantics=("arbitrary",))


def _pre_body(x_ref, deg_ref, w_ref, g_ref):
    dinv = lax.rsqrt(jnp.maximum(deg_ref[...], 1.0))
    g_ref[...] = dinv * jnp.dot(x_ref[...], w_ref[...],
                                preferred_element_type=jnp.float32)


def _tc_pre(x, deg, W):
    return pl.pallas_call(
        _pre_body,
        grid=(GRID,),
        in_specs=[
            pl.BlockSpec((BLK, D), lambda i: (i, 0)),
            pl.BlockSpec((BLK, 1), lambda i: (i, 0)),
            pl.BlockSpec((D, D), lambda i: (0, 0)),
        ],
        out_specs=pl.BlockSpec((BLK, D), lambda i: (i, 0)),
        out_shape=jax.ShapeDtypeStruct((N, D), jnp.float32),
        compiler_params=_tc_params,
    )(x, deg, W)


def _tdot(a, b):
    return lax.dot_general(a, b, (((0,), (0,)), ((), ())),
                           preferred_element_type=jnp.float32)


def _c1_body_factory(has_next, first):
    def body(*refs):
        if has_next:
            (s_ref, g_ref, deg_ref, batc_ref, b_ref, wc_ref, wp_ref,
             wn_ref) = refs[:8]
            outs = refs[8:]
        else:
            s_ref, g_ref, deg_ref, batc_ref, b_ref, wc_ref, wp_ref = refs[:7]
            outs = refs[7:]
        xn_ref = outs[0]
        idx = 1
        if has_next:
            gn_ref = outs[idx]
            idx += 1
        segc_ref = outs[idx]
        segp_ref = outs[idx + 1]
        idx += 2
        if first:
            cnt_ref = outs[idx]

        i = pl.program_id(0)
        dinv = lax.rsqrt(jnp.maximum(deg_ref[...], 1.0))
        xn = jnp.maximum((s_ref[...] + g_ref[...]) * dinv + b_ref[...], 0.0)
        xn_ref[...] = xn
        if has_next:
            gn_ref[...] = dinv * jnp.dot(xn, wn_ref[...],
                                         preferred_element_type=jnp.float32)
        iota_r = lax.broadcasted_iota(jnp.int32, (1, B), 1)
        mb = (batc_ref[...] == iota_r).astype(jnp.float32)
        hc = jnp.dot(xn, wc_ref[...], preferred_element_type=jnp.float32)
        hp = jnp.dot(xn, wp_ref[...], preferred_element_type=jnp.float32)
        cc = _tdot(mb, hc)
        cp = _tdot(mb, hp)
        if first:
            cnt = _tdot(mb, jnp.ones((BLK, 1), jnp.float32))

        @pl.when(i == 0)
        def _():
            segc_ref[...] = cc
            segp_ref[...] = cp
            if first:
                cnt_ref[...] = cnt

        @pl.when(i > 0)
        def _():
            segc_ref[...] += cc
            segp_ref[...] += cp
            if first:
                cnt_ref[...] += cnt

    return body


def _tc_c1(S, g, deg, batc, b_gcn, Wc, Wp, Wn, first):
    has_next = Wn is not None
    in_specs = [
        pl.BlockSpec((BLK, D), lambda i: (i, 0)),
        pl.BlockSpec((BLK, D), lambda i: (i, 0)),
        pl.BlockSpec((BLK, 1), lambda i: (i, 0)),
        pl.BlockSpec((BLK, 1), lambda i: (i, 0)),
        pl.BlockSpec((1, D), lambda i: (0, 0)),
        pl.BlockSpec((D, D), lambda i: (0, 0)),
        pl.BlockSpec((D, D), lambda i: (0, 0)),
    ]
    args = [S, g, deg, batc, b_gcn, Wc, Wp]
    if has_next:
        in_specs.append(pl.BlockSpec((D, D), lambda i: (0, 0)))
        args.append(Wn)
    out_specs = [pl.BlockSpec((BLK, D), lambda i: (i, 0))]
    out_shape = [jax.ShapeDtypeStruct((N, D), jnp.float32)]
    if has_next:
        out_specs.append(pl.BlockSpec((BLK, D), lambda i: (i, 0)))
        out_shape.append(jax.ShapeDtypeStruct((N, D), jnp.float32))
    out_specs += [pl.BlockSpec((B, D), lambda i: (0, 0)),
                  pl.BlockSpec((B, D), lambda i: (0, 0))]
    out_shape += [jax.ShapeDtypeStruct((B, D), jnp.float32),
                  jax.ShapeDtypeStruct((B, D), jnp.float32)]
    if first:
        out_specs.append(pl.BlockSpec((B, 1), lambda i: (0, 0)))
        out_shape.append(jax.ShapeDtypeStruct((B, 1), jnp.float32))
    return pl.pallas_call(
        _c1_body_factory(has_next, first),
        grid=(GRID,),
        in_specs=in_specs,
        out_specs=out_specs,
        out_shape=out_shape,
        compiler_params=_tc_params,
    )(*args)


def _c2_body(xn_ref, batc_ref, segc_ref, segp_ref, cnt_ref,
             pc_ref, pp_ref):
    i = pl.program_id(0)
    cntm = jnp.maximum(cnt_ref[...], 1.0)
    ctxc = jnp.tanh(segc_ref[...] / cntm)
    ctxp = jnp.tanh(segp_ref[...] / cntm)
    xn = xn_ref[...]
    batc = batc_ref[...]
    iota_r = lax.broadcasted_iota(jnp.int32, (1, B), 1)
    mb = (batc == iota_r).astype(jnp.float32)
    ec = jnp.dot(mb, ctxc, preferred_element_type=jnp.float32)
    ep = jnp.dot(mb, ctxp, preferred_element_type=jnp.float32)
    attc = jax.nn.sigmoid(jnp.sum(xn * ec, axis=1, keepdims=True))
    attp = jax.nn.sigmoid(jnp.sum(xn * ep, axis=1, keepdims=True))
    vc = _tdot(mb, xn * attc)
    vp = _tdot(mb, xn * attp)

    @pl.when(i == 0)
    def _():
        pc_ref[...] = vc
        pp_ref[...] = vp

    @pl.when(i > 0)
    def _():
        pc_ref[...] += vc
        pp_ref[...] += vp


def _tc_c2(xn, batc, segc, segp, cnt):
    return pl.pallas_call(
        _c2_body,
        grid=(GRID,),
        in_specs=[
            pl.BlockSpec((BLK, D), lambda i: (i, 0)),
            pl.BlockSpec((BLK, 1), lambda i: (i, 0)),
            pl.BlockSpec((B, D), lambda i: (0, 0)),
            pl.BlockSpec((B, D), lambda i: (0, 0)),
            pl.BlockSpec((B, 1), lambda i: (0, 0)),
        ],
        out_specs=[pl.BlockSpec((B, D), lambda i: (0, 0)),
                   pl.BlockSpec((B, D), lambda i: (0, 0))],
        out_shape=[jax.ShapeDtypeStruct((B, D), jnp.float32),
                   jax.ShapeDtypeStruct((B, D), jnp.float32)],
        compiler_params=_tc_params,
    )(xn, batc, segc, segp, cnt)


def _cos_rows(a, b):
    an = a / jnp.maximum(jnp.sqrt(jnp.sum(a * a, axis=1, keepdims=True)),
                         1e-8)
    bn = b / jnp.maximum(jnp.sqrt(jnp.sum(b * b, axis=1, keepdims=True)),
                         1e-8)
    return jnp.sum(an * bn, axis=1, keepdims=True)


def _corr_rows(a, b):
    ac = a - jnp.mean(a, axis=0, keepdims=True)
    bc = b - jnp.mean(b, axis=0, keepdims=True)
    c = _cos_rows(ac, bc)
    return c * c


def _final_body(pc10, pp10, pc11, pp11, pc12, pp12,
                pc20, pp20, pc21, pp21, pc22, pp22,
                wt_ref, vt_ref, bntn_ref, ws1_ref, bs1_ref, ws2_ref, bs2_ref,
                score_ref, loss_ref):
    com1 = [pc10[...], pc11[...], pc12[...]]
    pri1 = [pp10[...], pp11[...], pp12[...]]
    com2 = [pc20[...], pc21[...], pc22[...]]
    pri2 = [pp20[...], pp21[...], pp22[...]]
    f1 = jnp.concatenate([com1[2], pri1[2]], axis=1)
    f2 = jnp.concatenate([com2[2], pri2[2]], axis=1)
    cols = []
    for k in range(K):
        t = jnp.dot(f1, wt_ref[k], preferred_element_type=jnp.float32)
        cols.append(jnp.sum(t * f2, axis=1, keepdims=True))
    scoring = jnp.concatenate(cols, axis=1)
    blk = jnp.dot(jnp.concatenate([f1, f2], axis=1), vt_ref[...],
                  preferred_element_type=jnp.float32)
    sact = jnp.maximum(scoring + blk + bntn_ref[...], 0.0)
    h1 = jnp.maximum(jnp.dot(sact, ws1_ref[...],
                             preferred_element_type=jnp.float32)
                     + bs1_ref[...], 0.0)
    pre = jnp.sum(h1 * ws2_ref[...], axis=1, keepdims=True) + bs2_ref[...]
    score_ref[...] = jax.nn.sigmoid(pre)

    cor_sum = jnp.zeros((1, 1), jnp.float32)
    lratio_sum = jnp.zeros((1, 1), jnp.float32)
    for i in range(L):
        cor1 = _corr_rows(com1[i], pri1[i])
        cor2 = _corr_rows(com2[i], pri2[i])
        cor_sum = cor_sum + jnp.sum(cor1 + cor2, axis=0, keepdims=True)
        sim_com = jnp.exp(_cos_rows(com1[i], com2[i]))
        sim_pri = jnp.exp(_cos_rows(com2[i], pri2[i]))
        lr = jnp.log(sim_com / (sim_com + sim_pri))
        lratio_sum = lratio_sum + jnp.sum(lr, axis=0, keepdims=True)
    loss_ref[...] = (-lratio_sum / (L * B)) + 0.5 * (cor_sum / (L * B))


def _tc_final(pools, Wt, Vt, b_ntn, W_s1, b_s1, W_s2r, b_s2):
    def full(shp):
        return pl.BlockSpec(shp, lambda: tuple(0 for _ in shp))

    in_specs = [full((B, D)) for _ in range(12)] + [
        full((K, 2 * D, 2 * D)), full((4 * D, K)), full((1, K)),
        full((K, K)), full((1, K)), full((1, K)), full((1, 1)),
    ]
    return pl.pallas_call(
        _final_body,
        grid=(),
        in_specs=in_specs,
        out_specs=[full((B, 1)), full((1, 1))],
        out_shape=[jax.ShapeDtypeStruct((B, 1), jnp.float32),
                   jax.ShapeDtypeStruct((1, 1), jnp.float32)],
    )(*pools, Wt, Vt, b_ntn, W_s1, b_s1, W_s2r, b_s2)


def _pad_edges(ei):
    pads = EPT * NSUB - E
    src = jnp.concatenate([ei[0], jnp.zeros((pads,), jnp.int32)])
    dst = jnp.concatenate([ei[1], jnp.full((pads,), DUMP, jnp.int32)])
    return (src.reshape(NSUB, NCHUNK, CHUNK),
            dst.reshape(NSUB, NCHUNK, CHUNK))


def kernel(x1, x2, edge_index1, edge_index2, batch1, batch2,
           W_gcn0, b_gcn0, W_com0, W_pri0,
           W_gcn1, b_gcn1, W_com1, W_pri1,
           W_gcn2, b_gcn2, W_com2, W_pri2,
           W_ntn, V_ntn, b_ntn, W_s1, b_s1, W_s2, b_s2):
    src1p, dst1p = _pad_edges(edge_index1)
    src2p, dst2p = _pad_edges(edge_index2)
    oini = jnp.ones((NPAD, 16), jnp.float32)
    zini = jnp.zeros((NPAD, D), jnp.float32)

    degw1, degw2 = _sc_deg(dst1p, dst2p, oini)
    deg1 = degw1[:N, 0:1]
    deg2 = degw2[:N, 0:1]

    batc1 = batch1.reshape(N, 1)
    batc2 = batch2.reshape(N, 1)

    Wg = [W_gcn0, W_gcn1, W_gcn2]
    bg = [b_gcn0.reshape(1, D), b_gcn1.reshape(1, D), b_gcn2.reshape(1, D)]
    Wc = [W_com0, W_com1, W_com2]
    Wp = [W_pri0, W_pri1, W_pri2]

    g1 = _tc_pre(x1, deg1, Wg[0])
    g2 = _tc_pre(x2, deg2, Wg[0])

    pools1 = []
    pools2 = []
    cnt1 = cnt2 = None
    for i in range(L):
        S1p, S2p = _sc_msg(src1p, dst1p, src2p, dst2p, g1, g2, zini)
        S1 = S1p[:N]
        S2 = S2p[:N]
        Wn = Wg[i + 1] if i < L - 1 else None
        r1 = _tc_c1(S1, g1, deg1, batc1, bg[i], Wc[i], Wp[i], Wn,
                    first=(i == 0))
        r2 = _tc_c1(S2, g2, deg2, batc2, bg[i], Wc[i], Wp[i], Wn,
                    first=(i == 0))
        if i == 0:
            x1n, g1, segc1, segp1, cnt1 = r1
            x2n, g2, segc2, segp2, cnt2 = r2
        elif i < L - 1:
            x1n, g1, segc1, segp1 = r1
            x2n, g2, segc2, segp2 = r2
        else:
            x1n, segc1, segp1 = r1
            x2n, segc2, segp2 = r2
        pc1, pp1 = _tc_c2(x1n, batc1, segc1, segp1, cnt1)
        pc2, pp2 = _tc_c2(x2n, batc2, segc2, segp2, cnt2)
        pools1.append((pc1, pp1))
        pools2.append((pc2, pp2))

    pools = [pools1[0][0], pools1[0][1], pools1[1][0], pools1[1][1],
             pools1[2][0], pools1[2][1],
             pools2[0][0], pools2[0][1], pools2[1][0], pools2[1][1],
             pools2[2][0], pools2[2][1]]
    Wt = jnp.transpose(W_ntn, (2, 0, 1))
    Vt = V_ntn.T
    score, loss = _tc_final(pools, Wt, Vt, b_ntn.reshape(1, K),
                            W_s1, b_s1.reshape(1, K),
                            W_s2.reshape(1, K), b_s2.reshape(1, 1))
    return score, loss[0, 0]


# speedup vs baseline: 10.0851x; 1.0404x over previous
import jax
import jax.numpy as jnp
from jax import lax
from jax.experimental import pallas as pl
from jax.experimental.pallas import tpu as pltpu
from jax.experimental.pallas import tpu_sc as plsc

N = 10000
E = 320000
D = 128
B = 128
K = 16
L = 3

NSUB = 16
CHUNK = 64
NCHUNK = 320
EPT = CHUNK * NCHUNK
NPAD = 10112
DUMP = N
STRIPE = NPAD // NSUB
NBUF = 4
LA = 2

BLK = 2000
GRID = N // BLK

def _mesh():
    return plsc.VectorSubcoreMesh(core_axis_name="c", subcore_axis_name="s")


def _deg_body(dst1, dst2, oini, out1, out2, idx_d, ones_b, acc,
              s0, s1, s2, s3):
    c = lax.axis_index("c")
    s = lax.axis_index("s")
    sems = [s0, s1, s2, s3]

    def run(dst_r, out_r):
        pltpu.sync_copy(dst_r.at[s], idx_d)
        pltpu.sync_copy(oini.at[pl.ds(0, CHUNK)], ones_b)
        pltpu.sync_copy(oini.at[pl.ds(s * STRIPE, STRIPE)],
                        acc.at[pl.ds(s * STRIPE, STRIPE)])
        plsc.subcore_barrier()

        def outer(o, carry):
            for b in range(NBUF):
                t = o * NBUF + b

                @pl.when(t >= NBUF)
                def _():
                    pltpu.make_async_copy(
                        ones_b, acc.at[idx_d.at[0]], sems[b]).wait()

                pltpu.async_copy(ones_b, acc.at[idx_d.at[t]], sems[b],
                                 add=True)
            return carry

        lax.fori_loop(0, NCHUNK // NBUF, outer, 0)
        for b in range(NBUF):
            pltpu.make_async_copy(ones_b, acc.at[idx_d.at[0]], sems[b]).wait()
        plsc.subcore_barrier()
        pltpu.sync_copy(acc.at[pl.ds(s * STRIPE, STRIPE)],
                        out_r.at[pl.ds(s * STRIPE, STRIPE)])

    @pl.when(c == 0)
    def _():
        run(dst1, out1)

    @pl.when(c == 1)
    def _():
        run(dst2, out2)


def _sc_deg(dst1p, dst2p, oini):
    fn = pl.kernel(
        _deg_body,
        out_type=(jax.ShapeDtypeStruct((NPAD, 16), jnp.float32),
                  jax.ShapeDtypeStruct((NPAD, 16), jnp.float32)),
        mesh=_mesh(),
        scratch_types=[
            pltpu.VMEM((NCHUNK, CHUNK), jnp.int32),
            pltpu.VMEM((CHUNK, 16), jnp.float32),
            pltpu.VMEM_SHARED((NPAD, 16), jnp.float32),
            pltpu.SemaphoreType.DMA,
            pltpu.SemaphoreType.DMA,
            pltpu.SemaphoreType.DMA,
            pltpu.SemaphoreType.DMA,
        ],
    )
    return fn(dst1p, dst2p, oini)


IB = 8
NGRP = NCHUNK // IB
NIB = 3
NBUF = 4


def _msg_body(src1, dst1, src2, dst2, g1, g2, zini, out1, out2,
              idx_s, idx_d, rows, acc,
              sg0, sg1, sg2, sg3, ss0, ss1, ss2, ss3, si):
    c = lax.axis_index("c")
    s = lax.axis_index("s")
    sem_g = [sg0, sg1, sg2, sg3]
    sem_s = [ss0, ss1, ss2, ss3]

    def run(src_r, dst_r, g_r, out_r):
        pltpu.sync_copy(zini.at[pl.ds(s * STRIPE, STRIPE)],
                        acc.at[pl.ds(s * STRIPE, STRIPE)])
        pltpu.async_copy(src_r.at[s].at[pl.ds(0, IB)], idx_s.at[0], si)
        pltpu.async_copy(dst_r.at[s].at[pl.ds(0, IB)], idx_d.at[0], si)
        plsc.subcore_barrier()

        def wait_gather(b):
            pltpu.make_async_copy(g_r.at[idx_s.at[0].at[0]], rows.at[b],
                                  sem_g[b]).wait()

        def wait_scatter(b):
            pltpu.make_async_copy(rows.at[b], acc.at[idx_d.at[0].at[0]],
                                  sem_s[b]).wait()

        def wait_idx():
            pltpu.make_async_copy(src_r.at[s].at[pl.ds(0, IB)],
                                  idx_s.at[0], si).wait()

        def outer(g, carry):
            gb = g % NIB
            gbn = (g + 1) % NIB
            gbp = (g - 1) % NIB
            wait_idx()
            wait_idx()

            @pl.when(g + 1 < NGRP)
            def _():
                pltpu.async_copy(src_r.at[s].at[pl.ds((g + 1) * IB, IB)],
                                 idx_s.at[gbn], si)
                pltpu.async_copy(dst_r.at[s].at[pl.ds((g + 1) * IB, IB)],
                                 idx_d.at[gbn], si)

            for t in range(IB):
                u = g * IB + t
                b = t % NBUF
                bp = (t - 1) % NBUF

                @pl.when(u >= NBUF)
                def _():
                    wait_scatter(b)

                pltpu.async_copy(g_r.at[idx_s.at[gb].at[t]], rows.at[b],
                                 sem_g[b])

                @pl.when(u >= 1)
                def _():
                    wait_gather(bp)
                    if t > 0:
                        pidx = idx_d.at[gb].at[t - 1]
                    else:
                        pidx = idx_d.at[gbp].at[IB - 1]
                    pltpu.async_copy(rows.at[bp], acc.at[pidx],
                                     sem_s[bp], add=True)
            return carry

        lax.fori_loop(0, NGRP, outer, 0)
        bl = (IB - 1) % NBUF
        wait_gather(bl)
        pltpu.async_copy(rows.at[bl],
                         acc.at[idx_d.at[(NGRP - 1) % NIB].at[IB - 1]],
                         sem_s[bl], add=True)
        for b in range(NBUF):
            wait_scatter(b)
        plsc.subcore_barrier()
        pltpu.sync_copy(acc.at[pl.ds(s * STRIPE, STRIPE)],
                        out_r.at[pl.ds(s * STRIPE, STRIPE)])

    @pl.when(c == 0)
    def _():
        run(src1, dst1, g1, out1)

    @pl.when(c == 1)
    def _():
        run(src2, dst2, g2, out2)


def _sc_msg(src1p, dst1p, src2p, dst2p, g1, g2, zini):
    fn = pl.kernel(
        _msg_body,
        out_type=(jax.ShapeDtypeStruct((NPAD, D), jnp.float32),
                  jax.ShapeDtypeStruct((NPAD, D), jnp.float32)),
        mesh=_mesh(),
        scratch_types=[
            pltpu.VMEM((NIB, IB, CHUNK), jnp.int32),
            pltpu.VMEM((NIB, IB, CHUNK), jnp.int32),
            pltpu.VMEM((NBUF, CHUNK, D), jnp.float32),
            pltpu.VMEM_SHARED((NPAD, D), jnp.float32),
            pltpu.SemaphoreType.DMA, pltpu.SemaphoreType.DMA,
            pltpu.SemaphoreType.DMA, pltpu.SemaphoreType.DMA,
            pltpu.SemaphoreType.DMA, pltpu.SemaphoreType.DMA,
            pltpu.SemaphoreType.DMA, pltpu.SemaphoreType.DMA,
            pltpu.SemaphoreType.DMA,
        ],
    )
    return fn(src1p, dst1p, src2p, dst2p, g1, g2, zini)


_tc_params = pltpu.CompilerParams(dimension_semantics=("arbitrary",))


def _pre_body(x_ref, deg_ref, w_ref, g_ref):
    dinv = lax.rsqrt(jnp.maximum(deg_ref[...], 1.0))
    g_ref[...] = dinv * jnp.dot(x_ref[...], w_ref[...],
                                preferred_element_type=jnp.float32)


def _tc_pre(x, deg, W):
    return pl.pallas_call(
        _pre_body,
        grid=(GRID,),
        in_specs=[
            pl.BlockSpec((BLK, D), lambda i: (i, 0)),
            pl.BlockSpec((BLK, 1), lambda i: (i, 0)),
            pl.BlockSpec((D, D), lambda i: (0, 0)),
        ],
        out_specs=pl.BlockSpec((BLK, D), lambda i: (i, 0)),
        out_shape=jax.ShapeDtypeStruct((N, D), jnp.float32),
        compiler_params=_tc_params,
    )(x, deg, W)


def _tdot(a, b):
    return lax.dot_general(a, b, (((0,), (0,)), ((), ())),
                           preferred_element_type=jnp.float32)


def _c1_body_factory(has_next, first):
    def body(*refs):
        if has_next:
            (s_ref, g_ref, deg_ref, batc_ref, b_ref, wc_ref, wp_ref,
             wn_ref) = refs[:8]
            outs = refs[8:]
        else:
            s_ref, g_ref, deg_ref, batc_ref, b_ref, wc_ref, wp_ref = refs[:7]
            outs = refs[7:]
        xn_ref = outs[0]
        idx = 1
        if has_next:
            gn_ref = outs[idx]
            idx += 1
        segc_ref = outs[idx]
        segp_ref = outs[idx + 1]
        idx += 2
        if first:
            cnt_ref = outs[idx]

        i = pl.program_id(0)
        dinv = lax.rsqrt(jnp.maximum(deg_ref[...], 1.0))
        xn = jnp.maximum((s_ref[...] + g_ref[...]) * dinv + b_ref[...], 0.0)
        xn_ref[...] = xn
        if has_next:
            gn_ref[...] = dinv * jnp.dot(xn, wn_ref[...],
                                         preferred_element_type=jnp.float32)
        iota_r = lax.broadcasted_iota(jnp.int32, (1, B), 1)
        mb = (batc_ref[...] == iota_r).astype(jnp.float32)
        hc = jnp.dot(xn, wc_ref[...], preferred_element_type=jnp.float32)
        hp = jnp.dot(xn, wp_ref[...], preferred_element_type=jnp.float32)
        cc = _tdot(mb, hc)
        cp = _tdot(mb, hp)
        if first:
            cnt = _tdot(mb, jnp.ones((BLK, 1), jnp.float32))

        @pl.when(i == 0)
        def _():
            segc_ref[...] = cc
            segp_ref[...] = cp
            if first:
                cnt_ref[...] = cnt

        @pl.when(i > 0)
        def _():
            segc_ref[...] += cc
            segp_ref[...] += cp
            if first:
                cnt_ref[...] += cnt

    return body


def _tc_c1(S, g, deg, batc, b_gcn, Wc, Wp, Wn, first):
    has_next = Wn is not None
    in_specs = [
        pl.BlockSpec((BLK, D), lambda i: (i, 0)),
        pl.BlockSpec((BLK, D), lambda i: (i, 0)),
        pl.BlockSpec((BLK, 1), lambda i: (i, 0)),
        pl.BlockSpec((BLK, 1), lambda i: (i, 0)),
        pl.BlockSpec((1, D), lambda i: (0, 0)),
        pl.BlockSpec((D, D), lambda i: (0, 0)),
        pl.BlockSpec((D, D), lambda i: (0, 0)),
    ]
    args = [S, g, deg, batc, b_gcn, Wc, Wp]
    if has_next:
        in_specs.append(pl.BlockSpec((D, D), lambda i: (0, 0)))
        args.append(Wn)
    out_specs = [pl.BlockSpec((BLK, D), lambda i: (i, 0))]
    out_shape = [jax.ShapeDtypeStruct((N, D), jnp.float32)]
    if has_next:
        out_specs.append(pl.BlockSpec((BLK, D), lambda i: (i, 0)))
        out_shape.append(jax.ShapeDtypeStruct((N, D), jnp.float32))
    out_specs += [pl.BlockSpec((B, D), lambda i: (0, 0)),
                  pl.BlockSpec((B, D), lambda i: (0, 0))]
    out_shape += [jax.ShapeDtypeStruct((B, D), jnp.float32),
                  jax.ShapeDtypeStruct((B, D), jnp.float32)]
    if first:
        out_specs.append(pl.BlockSpec((B, 1), lambda i: (0, 0)))
        out_shape.append(jax.ShapeDtypeStruct((B, 1), jnp.float32))
    return pl.pallas_call(
        _c1_body_factory(has_next, first),
        grid=(GRID,),
        in_specs=in_specs,
        out_specs=out_specs,
        out_shape=out_shape,
        compiler_params=_tc_params,
    )(*args)


def _c2_body(xn_ref, batc_ref, segc_ref, segp_ref, cnt_ref,
             pc_ref, pp_ref):
    i = pl.program_id(0)
    cntm = jnp.maximum(cnt_ref[...], 1.0)
    ctxc = jnp.tanh(segc_ref[...] / cntm)
    ctxp = jnp.tanh(segp_ref[...] / cntm)
    xn = xn_ref[...]
    batc = batc_ref[...]
    iota_r = lax.broadcasted_iota(jnp.int32, (1, B), 1)
    mb = (batc == iota_r).astype(jnp.float32)
    ec = jnp.dot(mb, ctxc, preferred_element_type=jnp.float32)
    ep = jnp.dot(mb, ctxp, preferred_element_type=jnp.float32)
    attc = jax.nn.sigmoid(jnp.sum(xn * ec, axis=1, keepdims=True))
    attp = jax.nn.sigmoid(jnp.sum(xn * ep, axis=1, keepdims=True))
    vc = _tdot(mb, xn * attc)
    vp = _tdot(mb, xn * attp)

    @pl.when(i == 0)
    def _():
        pc_ref[...] = vc
        pp_ref[...] = vp

    @pl.when(i > 0)
    def _():
        pc_ref[...] += vc
        pp_ref[...] += vp


def _tc_c2(xn, batc, segc, segp, cnt):
    return pl.pallas_call(
        _c2_body,
        grid=(GRID,),
        in_specs=[
            pl.BlockSpec((BLK, D), lambda i: (i, 0)),
            pl.BlockSpec((BLK, 1), lambda i: (i, 0)),
            pl.BlockSpec((B, D), lambda i: (0, 0)),
            pl.BlockSpec((B, D), lambda i: (0, 0)),
            pl.BlockSpec((B, 1), lambda i: (0, 0)),
        ],
        out_specs=[pl.BlockSpec((B, D), lambda i: (0, 0)),
                   pl.BlockSpec((B, D), lambda i: (0, 0))],
        out_shape=[jax.ShapeDtypeStruct((B, D), jnp.float32),
                   jax.ShapeDtypeStruct((B, D), jnp.float32)],
        compiler_params=_tc_params,
    )(xn, batc, segc, segp, cnt)


def _cos_rows(a, b):
    an = a / jnp.maximum(jnp.sqrt(jnp.sum(a * a, axis=1, keepdims=True)),
                         1e-8)
    bn = b / jnp.maximum(jnp.sqrt(jnp.sum(b * b, axis=1, keepdims=True)),
                         1e-8)
    return jnp.sum(an * bn, axis=1, keepdims=True)


def _corr_rows(a, b):
    ac = a - jnp.mean(a, axis=0, keepdims=True)
    bc = b - jnp.mean(b, axis=0, keepdims=True)
    c = _cos_rows(ac, bc)
    return c * c


def _final_body(pc10, pp10, pc11, pp11, pc12, pp12,
                pc20, pp20, pc21, pp21, pc22, pp22,
                wt_ref, vt_ref, bntn_ref, ws1_ref, bs1_ref, ws2_ref, bs2_ref,
                score_ref, loss_ref):
    com1 = [pc10[...], pc11[...], pc12[...]]
    pri1 = [pp10[...], pp11[...], pp12[...]]
    com2 = [pc20[...], pc21[...], pc22[...]]
    pri2 = [pp20[...], pp21[...], pp22[...]]
    f1 = jnp.concatenate([com1[2], pri1[2]], axis=1)
    f2 = jnp.concatenate([com2[2], pri2[2]], axis=1)
    cols = []
    for k in range(K):
        t = jnp.dot(f1, wt_ref[k], preferred_element_type=jnp.float32)
        cols.append(jnp.sum(t * f2, axis=1, keepdims=True))
    scoring = jnp.concatenate(cols, axis=1)
    blk = jnp.dot(jnp.concatenate([f1, f2], axis=1), vt_ref[...],
                  preferred_element_type=jnp.float32)
    sact = jnp.maximum(scoring + blk + bntn_ref[...], 0.0)
    h1 = jnp.maximum(jnp.dot(sact, ws1_ref[...],
                             preferred_element_type=jnp.float32)
                     + bs1_ref[...], 0.0)
    pre = jnp.sum(h1 * ws2_ref[...], axis=1, keepdims=True) + bs2_ref[...]
    score_ref[...] = jax.nn.sigmoid(pre)

    cor_sum = jnp.zeros((1, 1), jnp.float32)
    lratio_sum = jnp.zeros((1, 1), jnp.float32)
    for i in range(L):
        cor1 = _corr_rows(com1[i], pri1[i])
        cor2 = _corr_rows(com2[i], pri2[i])
        cor_sum = cor_sum + jnp.sum(cor1 + cor2, axis=0, keepdims=True)
        sim_com = jnp.exp(_cos_rows(com1[i], com2[i]))
        sim_pri = jnp.exp(_cos_rows(com2[i], pri2[i]))
        lr = jnp.log(sim_com / (sim_com + sim_pri))
        lratio_sum = lratio_sum + jnp.sum(lr, axis=0, keepdims=True)
    loss_ref[...] = (-lratio_sum / (L * B)) + 0.5 * (cor_sum / (L * B))


def _tc_final(pools, Wt, Vt, b_ntn, W_s1, b_s1, W_s2r, b_s2):
    def full(shp):
        return pl.BlockSpec(shp, lambda: tuple(0 for _ in shp))

    in_specs = [full((B, D)) for _ in range(12)] + [
        full((K, 2 * D, 2 * D)), full((4 * D, K)), full((1, K)),
        full((K, K)), full((1, K)), full((1, K)), full((1, 1)),
    ]
    return pl.pallas_call(
        _final_body,
        grid=(),
        in_specs=in_specs,
        out_specs=[full((B, 1)), full((1, 1))],
        out_shape=[jax.ShapeDtypeStruct((B, 1), jnp.float32),
                   jax.ShapeDtypeStruct((1, 1), jnp.float32)],
    )(*pools, Wt, Vt, b_ntn, W_s1, b_s1, W_s2r, b_s2)


def _pad_edges(ei):
    pads = EPT * NSUB - E
    src = jnp.concatenate([ei[0], jnp.zeros((pads,), jnp.int32)])
    dst = jnp.concatenate([ei[1], jnp.full((pads,), DUMP, jnp.int32)])
    return (src.reshape(NSUB, NCHUNK, CHUNK),
            dst.reshape(NSUB, NCHUNK, CHUNK))


def kernel(x1, x2, edge_index1, edge_index2, batch1, batch2,
           W_gcn0, b_gcn0, W_com0, W_pri0,
           W_gcn1, b_gcn1, W_com1, W_pri1,
           W_gcn2, b_gcn2, W_com2, W_pri2,
           W_ntn, V_ntn, b_ntn, W_s1, b_s1, W_s2, b_s2):
    src1p, dst1p = _pad_edges(edge_index1)
    src2p, dst2p = _pad_edges(edge_index2)
    oini = jnp.ones((NPAD, 16), jnp.float32)
    zini = jnp.zeros((NPAD, D), jnp.float32)

    degw1, degw2 = _sc_deg(dst1p, dst2p, oini)
    deg1 = degw1[:N, 0:1]
    deg2 = degw2[:N, 0:1]

    batc1 = batch1.reshape(N, 1)
    batc2 = batch2.reshape(N, 1)

    Wg = [W_gcn0, W_gcn1, W_gcn2]
    bg = [b_gcn0.reshape(1, D), b_gcn1.reshape(1, D), b_gcn2.reshape(1, D)]
    Wc = [W_com0, W_com1, W_com2]
    Wp = [W_pri0, W_pri1, W_pri2]

    g1 = _tc_pre(x1, deg1, Wg[0])
    g2 = _tc_pre(x2, deg2, Wg[0])

    pools1 = []
    pools2 = []
    cnt1 = cnt2 = None
    for i in range(L):
        S1p, S2p = _sc_msg(src1p, dst1p, src2p, dst2p, g1, g2, zini)
        S1 = S1p[:N]
        S2 = S2p[:N]
        Wn = Wg[i + 1] if i < L - 1 else None
        r1 = _tc_c1(S1, g1, deg1, batc1, bg[i], Wc[i], Wp[i], Wn,
                    first=(i == 0))
        r2 = _tc_c1(S2, g2, deg2, batc2, bg[i], Wc[i], Wp[i], Wn,
                    first=(i == 0))
        if i == 0:
            x1n, g1, segc1, segp1, cnt1 = r1
            x2n, g2, segc2, segp2, cnt2 = r2
        elif i < L - 1:
            x1n, g1, segc1, segp1 = r1
            x2n, g2, segc2, segp2 = r2
        else:
            x1n, segc1, segp1 = r1
            x2n, segc2, segp2 = r2
        pc1, pp1 = _tc_c2(x1n, batc1, segc1, segp1, cnt1)
        pc2, pp2 = _tc_c2(x2n, batc2, segc2, segp2, cnt2)
        pools1.append((pc1, pp1))
        pools2.append((pc2, pp2))

    pools = [pools1[0][0], pools1[0][1], pools1[1][0], pools1[1][1],
             pools1[2][0], pools1[2][1],
             pools2[0][0], pools2[0][1], pools2[1][0], pools2[1][1],
             pools2[2][0], pools2[2][1]]
    Wt = jnp.transpose(W_ntn, (2, 0, 1))
    Vt = V_ntn.T
    score, loss = _tc_final(pools, Wt, Vt, b_ntn.reshape(1, K),
                            W_s1, b_s1.reshape(1, K),
                            W_s2.reshape(1, K), b_s2.reshape(1, 1))
    return score, loss[0, 0]
